# Initial kernel scaffold; baseline (speedup 1.0000x reference)
import jax
import jax.numpy as jnp
from jax.experimental import pallas as pl


def kernel(h_src, h_dst, edge_index, W_src, W_dst, a, W_ih, W_hh, W_out):
    raise NotImplementedError("write your pallas kernel here")


# speedup vs baseline: 12.5862x; 12.5862x over previous
import functools

import jax
import jax.numpy as jnp
from jax import lax
from jax.experimental import pallas as pl
from jax.experimental.pallas import tpu as pltpu
from jax.experimental.pallas import tpu_sc as plsc

N = 10000
E = 320000
H = 128
NC = 2
NS = 16
NW = NC * NS
EPW = E // NW
CHUNK = 80
NCHUNKS = EPW // CHUNK
GROUPS = CHUNK // 16
NZCH = N // CHUNK
B1 = 1000

NEG_SLOPE = 0.2


def _lrelu(x):
    return jnp.where(x >= 0, x, NEG_SLOPE * x)


def _prep_body(hs_ref, hd_ref, ws_ref, wd_ref, a_ref,
               zs_ref, es_ref, ed_ref, mx_ref):
    i = pl.program_id(0)
    zs = jnp.dot(hs_ref[...], ws_ref[...], preferred_element_type=jnp.float32)
    zs_ref[...] = zs
    es = jnp.dot(zs, a_ref[...], preferred_element_type=jnp.float32, precision=lax.Precision.HIGHEST)
    es_ref[...] = es
    zd = jnp.dot(hd_ref[...], wd_ref[...], preferred_element_type=jnp.float32)
    ed_ref[...] = jnp.dot(zd, a_ref[...], preferred_element_type=jnp.float32, precision=lax.Precision.HIGHEST)

    @pl.when(i == 0)
    def _():
        mx_ref[...] = jnp.full((1, 1), -jnp.inf, jnp.float32)

    mx_ref[...] = jnp.maximum(mx_ref[...], jnp.max(es))


def _prep(h_src, h_dst, W_src, W_dst, a2):
    return pl.pallas_call(
        _prep_body,
        grid=(N // B1,),
        in_specs=[
            pl.BlockSpec((B1, H), lambda i: (i, 0)),
            pl.BlockSpec((B1, H), lambda i: (i, 0)),
            pl.BlockSpec((H, H), lambda i: (0, 0)),
            pl.BlockSpec((H, H), lambda i: (0, 0)),
            pl.BlockSpec((H, 1), lambda i: (0, 0)),
        ],
        out_specs=[
            pl.BlockSpec((B1, H), lambda i: (i, 0)),
            pl.BlockSpec((B1, 1), lambda i: (i, 0)),
            pl.BlockSpec((B1, 1), lambda i: (i, 0)),
            pl.BlockSpec((1, 1), lambda i: (0, 0)),
        ],
        out_shape=[
            jax.ShapeDtypeStruct((N, H), jnp.float32),
            jax.ShapeDtypeStruct((N, 1), jnp.float32),
            jax.ShapeDtypeStruct((N, 1), jnp.float32),
            jax.ShapeDtypeStruct((1, 1), jnp.float32),
        ],
    )(h_src, h_dst, W_src, W_dst, a2)


def _sc_body(zs_hbm, es_hbm, ed_hbm, mx_hbm, src_hbm, dst_hbm,
             u_out, den_out,
             es_v, ed_v, sidx, didx, wbuf, rows,
             mx_v, u_acc, den_acc, sem):
    c = lax.axis_index("c")
    s = lax.axis_index("s")
    wid = c * NS + s
    ebase = wid * EPW

    pltpu.sync_copy(es_hbm, es_v)
    pltpu.sync_copy(ed_hbm, ed_v)
    pltpu.sync_copy(mx_hbm, mx_v)

    def zfill(i, _):
        for h in range(8):
            rows[i, pl.ds(h * 16, 16)] = jnp.zeros((16,), jnp.float32)
        return 0

    lax.fori_loop(0, CHUNK, zfill, 0)
    for g in range(GROUPS):
        wbuf[pl.ds(g * 16, 16)] = jnp.zeros((16,), jnp.float32)

    for k in range(8):
        ci = s + NS * k

        @pl.when(ci < NZCH)
        def _():
            pltpu.sync_copy(rows, u_acc.at[pl.ds(ci * CHUNK, CHUNK)])
            pltpu.sync_copy(wbuf, den_acc.at[pl.ds(ci * CHUNK, CHUNK)])

    plsc.subcore_barrier()

    mx = mx_v[...]

    def chunk_body(j, _):
        pltpu.sync_copy(src_hbm.at[pl.ds(ebase + j * CHUNK, CHUNK)], sidx)
        pltpu.sync_copy(dst_hbm.at[pl.ds(ebase + j * CHUNK, CHUNK)], didx)
        for g in range(GROUPS):
            s16 = sidx[pl.ds(g * 16, 16)]
            d16 = didx[pl.ds(g * 16, 16)]
            es_s = plsc.load_gather(es_v, [s16])
            ed_d = plsc.load_gather(ed_v, [d16])
            w = jnp.exp(_lrelu(es_s - ed_d) - _lrelu(mx - ed_d))
            wbuf[pl.ds(g * 16, 16)] = w

        pltpu.async_copy(zs_hbm.at[sidx], rows, sem).wait()

        def srow(i, _):
            wv = plsc.load_gather(wbuf, [jnp.broadcast_to(i, (16,))])
            for h in range(8):
                rows[i, pl.ds(h * 16, 16)] = rows[i, pl.ds(h * 16, 16)] * wv
            return 0

        lax.fori_loop(0, CHUNK, srow, 0)

        pltpu.sync_copy(rows, u_acc.at[didx], add=True)
        pltpu.sync_copy(wbuf, den_acc.at[didx], add=True)
        return 0

    lax.fori_loop(0, NCHUNKS, chunk_body, 0)

    plsc.subcore_barrier()

    for k in range(8):
        ci = s + NS * k

        @pl.when(ci < NZCH)
        def _():
            pltpu.sync_copy(u_acc.at[pl.ds(ci * CHUNK, CHUNK)], rows)
            pltpu.sync_copy(rows, u_out.at[pl.ds(c * N + ci * CHUNK, CHUNK)])
            pltpu.sync_copy(den_acc.at[pl.ds(ci * CHUNK, CHUNK)], wbuf)
            pltpu.sync_copy(wbuf, den_out.at[pl.ds(c * N + ci * CHUNK, CHUNK)])


_edge_phase = functools.partial(
    pl.kernel,
    out_type=[
        jax.ShapeDtypeStruct((NC * N, H), jnp.float32),
        jax.ShapeDtypeStruct((NC * N,), jnp.float32),
    ],
    mesh=plsc.VectorSubcoreMesh(core_axis_name="c", subcore_axis_name="s"),
    compiler_params=pltpu.CompilerParams(needs_layout_passes=False),
    scratch_types=[
        pltpu.VMEM((N,), jnp.float32),
        pltpu.VMEM((N,), jnp.float32),
        pltpu.VMEM((CHUNK,), jnp.int32),
        pltpu.VMEM((CHUNK,), jnp.int32),
        pltpu.VMEM((CHUNK,), jnp.float32),
        pltpu.VMEM((CHUNK, H), jnp.float32),
        pltpu.VMEM((16,), jnp.float32),
        pltpu.VMEM_SHARED((N, H), jnp.float32),
        pltpu.VMEM_SHARED((N,), jnp.float32),
        pltpu.SemaphoreType.DMA,
    ],
)(_sc_body)


def _gru_body(u_ref, den_ref, wz_ref, wn_ref, wo_ref, out_ref):
    usum = u_ref[0] + u_ref[1]
    den = den_ref[0] + den_ref[1]
    h_k = usum * (1.0 / jnp.maximum(den, 1e-30))
    zg = jax.nn.sigmoid(jnp.dot(h_k, wz_ref[...], preferred_element_type=jnp.float32))
    ng = jnp.tanh(jnp.dot(h_k, wn_ref[...], preferred_element_type=jnp.float32))
    out_ref[...] = jnp.dot((1.0 - zg) * ng, wo_ref[...],
                           preferred_element_type=jnp.float32)


def _gru(u, den, WzT, WnT, WoT):
    return pl.pallas_call(
        _gru_body,
        grid=(N // B1,),
        in_specs=[
            pl.BlockSpec((NC, B1, H), lambda i: (0, i, 0)),
            pl.BlockSpec((NC, B1, 1), lambda i: (0, i, 0)),
            pl.BlockSpec((H, H), lambda i: (0, 0)),
            pl.BlockSpec((H, H), lambda i: (0, 0)),
            pl.BlockSpec((H, H), lambda i: (0, 0)),
        ],
        out_specs=pl.BlockSpec((B1, H), lambda i: (i, 0)),
        out_shape=jax.ShapeDtypeStruct((N, H), jnp.float32),
    )(u, den, WzT, WnT, WoT)


def kernel(h_src, h_dst, edge_index, W_src, W_dst, a, W_ih, W_hh, W_out):
    ei = edge_index.astype(jnp.int32)
    src = ei[0]
    dst = ei[1]
    a2 = a.reshape(H, 1)
    zs, es2, ed2, mx = _prep(h_src, h_dst, W_src, W_dst, a2)
    es = es2.reshape(N)
    ed = ed2.reshape(N)
    mx16 = jnp.broadcast_to(mx.reshape(1), (16,))
    u, den = _edge_phase(zs, es, ed, mx16, src, dst)
    WzT = W_ih[H:2 * H].T
    WnT = W_ih[2 * H:3 * H].T
    WoT = W_out.T
    return _gru(u.reshape(NC, N, H), den.reshape(NC, N, 1), WzT, WnT, WoT)


# speedup vs baseline: 26.0057x; 2.0662x over previous
import functools

import jax
import jax.numpy as jnp
from jax import lax
from jax.experimental import pallas as pl
from jax.experimental.pallas import tpu as pltpu
from jax.experimental.pallas import tpu_sc as plsc

N = 10000
E = 320000
H = 128
NC = 2
NS = 16
NW = NC * NS
EPW = E // NW
CHUNK = 80
NCHUNKS = EPW // CHUNK
GROUPS = CHUNK // 16
NZCH = N // CHUNK
B1 = 1000

NEG_SLOPE = 0.2


def _lrelu(x):
    return jnp.where(x >= 0, x, NEG_SLOPE * x)


def _prep_body(hs_ref, hd_ref, ws_ref, wd_ref, a_ref,
               zs_ref, es_ref, ed_ref, mx_ref):
    i = pl.program_id(0)
    zs = jnp.dot(hs_ref[...], ws_ref[...], preferred_element_type=jnp.float32)
    zs_ref[...] = zs
    es = jnp.dot(zs, a_ref[...], preferred_element_type=jnp.float32, precision=lax.Precision.HIGHEST)
    es_ref[...] = es
    zd = jnp.dot(hd_ref[...], wd_ref[...], preferred_element_type=jnp.float32)
    ed_ref[...] = jnp.dot(zd, a_ref[...], preferred_element_type=jnp.float32, precision=lax.Precision.HIGHEST)

    @pl.when(i == 0)
    def _():
        mx_ref[...] = jnp.full((1, 1), -jnp.inf, jnp.float32)

    mx_ref[...] = jnp.maximum(mx_ref[...], jnp.max(es))


def _prep(h_src, h_dst, W_src, W_dst, a2):
    return pl.pallas_call(
        _prep_body,
        grid=(N // B1,),
        in_specs=[
            pl.BlockSpec((B1, H), lambda i: (i, 0)),
            pl.BlockSpec((B1, H), lambda i: (i, 0)),
            pl.BlockSpec((H, H), lambda i: (0, 0)),
            pl.BlockSpec((H, H), lambda i: (0, 0)),
            pl.BlockSpec((H, 1), lambda i: (0, 0)),
        ],
        out_specs=[
            pl.BlockSpec((B1, H), lambda i: (i, 0)),
            pl.BlockSpec((B1, 1), lambda i: (i, 0)),
            pl.BlockSpec((B1, 1), lambda i: (i, 0)),
            pl.BlockSpec((1, 1), lambda i: (0, 0)),
        ],
        out_shape=[
            jax.ShapeDtypeStruct((N, H), jnp.float32),
            jax.ShapeDtypeStruct((N, 1), jnp.float32),
            jax.ShapeDtypeStruct((N, 1), jnp.float32),
            jax.ShapeDtypeStruct((1, 1), jnp.float32),
        ],
    )(h_src, h_dst, W_src, W_dst, a2)


def _sc_body(zs_hbm, es_hbm, ed_hbm, mx_hbm, ep_hbm,
             u_out, den_out,
             eb0, eb1, eb2, eb3, di0, di1, di2, di3,
             eg0, eg1, eg2, eg3, dg0, dg1, dg2, dg3,
             wb0, wb1, wb2, wb3, rw0, rw1, rw2, rw3,
             mx_v, u_acc, den_acc,
             is0, is1, is2, is3, gs0, gs1, gs2, gs3,
             es0, es1, es2, es3, ss0, ss1, ss2, ss3):
    c = lax.axis_index("c")
    s = lax.axis_index("s")
    wid = c * NS + s
    cbase = wid * NCHUNKS

    ebuf = [eb0, eb1, eb2, eb3]
    didx = [di0, di1, di2, di3]
    esg = [eg0, eg1, eg2, eg3]
    edg = [dg0, dg1, dg2, dg3]
    wbuf = [wb0, wb1, wb2, wb3]
    rows = [rw0, rw1, rw2, rw3]
    isem = [is0, is1, is2, is3]
    gsem = [gs0, gs1, gs2, gs3]
    esem = [es0, es1, es2, es3]
    ssem = [ss0, ss1, ss2, ss3]

    pltpu.sync_copy(mx_hbm, mx_v)

    def zfill(i, _):
        for h in range(8):
            rw0[i, pl.ds(h * 16, 16)] = jnp.zeros((16,), jnp.float32)
        return 0

    lax.fori_loop(0, CHUNK, zfill, 0)
    for g in range(GROUPS):
        wb0[pl.ds(g * 16, 16)] = jnp.zeros((16,), jnp.float32)

    for k in range(8):
        ci = s + NS * k

        @pl.when(ci < NZCH)
        def _():
            pltpu.sync_copy(rw0, u_acc.at[pl.ds(ci * CHUNK, CHUNK)])
            pltpu.sync_copy(wb0, den_acc.at[pl.ds(ci * CHUNK, CHUNK)])

    plsc.subcore_barrier()

    mx = mx_v[...]

    def idx_start(j, b):
        pltpu.async_copy(ep_hbm.at[pl.ds((cbase + j) * 2 * CHUNK, 2 * CHUNK)],
                         ebuf[b], isem[b])

    def idx_wait(j, b):
        pltpu.make_async_copy(
            ep_hbm.at[pl.ds((cbase + j) * 2 * CHUNK, 2 * CHUNK)],
            ebuf[b], isem[b]).wait()

    def gathers_start(b):
        src_ix = ebuf[b].at[pl.ds(0, CHUNK)]
        dst_ix = ebuf[b].at[pl.ds(CHUNK, CHUNK)]
        pltpu.async_copy(es_hbm.at[src_ix], esg[b], esem[b])
        pltpu.async_copy(ed_hbm.at[dst_ix], edg[b], esem[b])
        pltpu.async_copy(zs_hbm.at[src_ix], rows[b], gsem[b])

    def esed_wait(b):
        src_ix = ebuf[b].at[pl.ds(0, CHUNK)]
        dst_ix = ebuf[b].at[pl.ds(CHUNK, CHUNK)]
        pltpu.make_async_copy(es_hbm.at[src_ix], esg[b], esem[b]).wait()
        pltpu.make_async_copy(ed_hbm.at[dst_ix], edg[b], esem[b]).wait()

    def rows_wait(b):
        src_ix = ebuf[b].at[pl.ds(0, CHUNK)]
        pltpu.make_async_copy(zs_hbm.at[src_ix], rows[b], gsem[b]).wait()

    def scatter_start(b):
        pltpu.async_copy(rows[b], u_acc.at[didx[b]], ssem[b], add=True)
        pltpu.async_copy(wbuf[b], den_acc.at[didx[b]], ssem[b], add=True)

    def scatter_wait(b):
        pltpu.make_async_copy(rows[b], u_acc.at[didx[b]], ssem[b]).wait()
        pltpu.make_async_copy(wbuf[b], den_acc.at[didx[b]], ssem[b]).wait()

    def process(b):
        esed_wait(b)
        for g in range(GROUPS):
            didx[b][pl.ds(g * 16, 16)] = ebuf[b][pl.ds(CHUNK + g * 16, 16)]
            es_s = esg[b][pl.ds(g * 16, 16)]
            ed_d = edg[b][pl.ds(g * 16, 16)]
            w = jnp.exp(_lrelu(es_s - ed_d) - _lrelu(mx - ed_d))
            wbuf[b][pl.ds(g * 16, 16)] = w
        rows_wait(b)

        def srow(i, _):
            wv = plsc.load_gather(wbuf[b], [jnp.broadcast_to(i, (16,))])
            for h in range(8):
                rows[b][i, pl.ds(h * 16, 16)] = rows[b][i, pl.ds(h * 16, 16)] * wv
            return 0

        lax.fori_loop(0, CHUNK, srow, 0)
        scatter_start(b)

    for j in range(3):
        idx_start(j, j)
        idx_wait(j, j)
    gathers_start(0)
    gathers_start(1)

    def quad(t, _):
        for i in range(4):
            jc = 4 * t + i
            process(i)

            @pl.when(jc >= 1)
            def _():
                scatter_wait((i + 3) % 4)

            @pl.when(jc <= NCHUNKS - 4)
            def _():
                idx_start(jc + 3, (i + 3) % 4)

            @pl.when(jc <= NCHUNKS - 3)
            def _():
                @pl.when(jc >= 1)
                def _():
                    idx_wait(jc + 2, (i + 2) % 4)

                gathers_start((i + 2) % 4)
        return 0

    lax.fori_loop(0, (NCHUNKS - 1) // 4, quad, 0)

    process(0)
    scatter_wait(3)
    scatter_wait(0)

    plsc.subcore_barrier()

    for k in range(8):
        ci = s + NS * k

        @pl.when(ci < NZCH)
        def _():
            pltpu.sync_copy(u_acc.at[pl.ds(ci * CHUNK, CHUNK)], rw0)
            pltpu.sync_copy(rw0, u_out.at[pl.ds(c * N + ci * CHUNK, CHUNK)])
            pltpu.sync_copy(den_acc.at[pl.ds(ci * CHUNK, CHUNK)], wb0)
            pltpu.sync_copy(wb0, den_out.at[pl.ds(c * N + ci * CHUNK, CHUNK)])


_edge_phase = functools.partial(
    pl.kernel,
    out_type=[
        jax.ShapeDtypeStruct((NC * N, H), jnp.float32),
        jax.ShapeDtypeStruct((NC * N,), jnp.float32),
    ],
    mesh=plsc.VectorSubcoreMesh(core_axis_name="c", subcore_axis_name="s"),
    compiler_params=pltpu.CompilerParams(needs_layout_passes=False),
    scratch_types=(
        [pltpu.VMEM((2 * CHUNK,), jnp.int32) for _ in range(4)]
        + [pltpu.VMEM((CHUNK,), jnp.int32) for _ in range(4)]
        + [pltpu.VMEM((CHUNK,), jnp.float32) for _ in range(4)]
        + [pltpu.VMEM((CHUNK,), jnp.float32) for _ in range(4)]
        + [pltpu.VMEM((CHUNK,), jnp.float32) for _ in range(4)]
        + [pltpu.VMEM((CHUNK, H), jnp.float32) for _ in range(4)]
        + [
            pltpu.VMEM((16,), jnp.float32),
            pltpu.VMEM_SHARED((N, H), jnp.float32),
            pltpu.VMEM_SHARED((N,), jnp.float32),
        ]
        + [pltpu.SemaphoreType.DMA] * 16
    ),
)(_sc_body)


def _gru_body(u_ref, den_ref, wz_ref, wn_ref, wo_ref, out_ref):
    usum = u_ref[0] + u_ref[1]
    den = den_ref[0] + den_ref[1]
    h_k = usum * (1.0 / jnp.maximum(den, 1e-30))
    zg = jax.nn.sigmoid(jnp.dot(h_k, wz_ref[...], preferred_element_type=jnp.float32))
    ng = jnp.tanh(jnp.dot(h_k, wn_ref[...], preferred_element_type=jnp.float32))
    out_ref[...] = jnp.dot((1.0 - zg) * ng, wo_ref[...],
                           preferred_element_type=jnp.float32)


def _gru(u, den, WzT, WnT, WoT):
    return pl.pallas_call(
        _gru_body,
        grid=(N // B1,),
        in_specs=[
            pl.BlockSpec((NC, B1, H), lambda i: (0, i, 0)),
            pl.BlockSpec((NC, B1, 1), lambda i: (0, i, 0)),
            pl.BlockSpec((H, H), lambda i: (0, 0)),
            pl.BlockSpec((H, H), lambda i: (0, 0)),
            pl.BlockSpec((H, H), lambda i: (0, 0)),
        ],
        out_specs=pl.BlockSpec((B1, H), lambda i: (i, 0)),
        out_shape=jax.ShapeDtypeStruct((N, H), jnp.float32),
    )(u, den, WzT, WnT, WoT)


def kernel(h_src, h_dst, edge_index, W_src, W_dst, a, W_ih, W_hh, W_out):
    ei = edge_index.astype(jnp.int32)
    src3 = ei[0].reshape(NW, NCHUNKS, CHUNK)
    dst3 = ei[1].reshape(NW, NCHUNKS, CHUNK)
    epack = jnp.stack([src3, dst3], axis=2).reshape(E * 2)
    a2 = a.reshape(H, 1)
    zs, es2, ed2, mx = _prep(h_src, h_dst, W_src, W_dst, a2)
    es = es2.reshape(N)
    ed = ed2.reshape(N)
    mx16 = jnp.broadcast_to(mx.reshape(1), (16,))
    u, den = _edge_phase(zs, es, ed, mx16, epack)
    WzT = W_ih[H:2 * H].T
    WnT = W_ih[2 * H:3 * H].T
    WoT = W_out.T
    return _gru(u.reshape(NC, N, H), den.reshape(NC, N, 1), WzT, WnT, WoT)


# speedup vs baseline: 29.6764x; 1.1412x over previous
import functools

import jax
import jax.numpy as jnp
from jax import lax
from jax.experimental import pallas as pl
from jax.experimental.pallas import tpu as pltpu
from jax.experimental.pallas import tpu_sc as plsc

N = 10000
E = 320000
H = 128
NC = 2
NS = 16
NW = NC * NS
EPW = E // NW
CHUNK = 80
NCHUNKS = EPW // CHUNK
GROUPS = CHUNK // 16
NZCH = N // CHUNK
B1 = 1000

NEG_SLOPE = 0.2


def _lrelu(x):
    return jnp.where(x >= 0, x, NEG_SLOPE * x)


def _prep_body(hs_ref, hd_ref, ws_ref, wd_ref, a_ref,
               zs_ref, es_ref, ed_ref, mx_ref):
    i = pl.program_id(0)
    zs = jnp.dot(hs_ref[...], ws_ref[...], preferred_element_type=jnp.float32)
    zs_ref[...] = zs
    es = jnp.dot(zs, a_ref[...], preferred_element_type=jnp.float32, precision=lax.Precision.HIGHEST)
    es_ref[...] = es
    zd = jnp.dot(hd_ref[...], wd_ref[...], preferred_element_type=jnp.float32)
    ed_ref[...] = jnp.dot(zd, a_ref[...], preferred_element_type=jnp.float32, precision=lax.Precision.HIGHEST)

    @pl.when(i == 0)
    def _():
        mx_ref[...] = jnp.full((1, 1), -jnp.inf, jnp.float32)

    mx_ref[...] = jnp.maximum(mx_ref[...], jnp.max(es))


def _prep(h_src, h_dst, W_src, W_dst, a2):
    return pl.pallas_call(
        _prep_body,
        grid=(N // B1,),
        in_specs=[
            pl.BlockSpec((B1, H), lambda i: (i, 0)),
            pl.BlockSpec((B1, H), lambda i: (i, 0)),
            pl.BlockSpec((H, H), lambda i: (0, 0)),
            pl.BlockSpec((H, H), lambda i: (0, 0)),
            pl.BlockSpec((H, 1), lambda i: (0, 0)),
        ],
        out_specs=[
            pl.BlockSpec((B1, H), lambda i: (i, 0)),
            pl.BlockSpec((B1, 1), lambda i: (i, 0)),
            pl.BlockSpec((B1, 1), lambda i: (i, 0)),
            pl.BlockSpec((1, 1), lambda i: (0, 0)),
        ],
        out_shape=[
            jax.ShapeDtypeStruct((N, H), jnp.float32),
            jax.ShapeDtypeStruct((N, 1), jnp.float32),
            jax.ShapeDtypeStruct((N, 1), jnp.float32),
            jax.ShapeDtypeStruct((1, 1), jnp.float32),
        ],
    )(h_src, h_dst, W_src, W_dst, a2)


def _sc_body(zs_hbm, es_hbm, ed_hbm, mx_hbm, ep_hbm,
             u_out, den_out,
             eb0, eb1, eb2, eb3, di0, di1, di2, di3,
             eg0, eg1, eg2, eg3, dg0, dg1, dg2, dg3,
             wb0, wb1, wb2, wb3, rw0, rw1, rw2, rw3,
             mx_v, u_acc, den_acc,
             is0, is1, is2, is3, gs0, gs1, gs2, gs3,
             es0, es1, es2, es3, ss0, ss1, ss2, ss3):
    c = lax.axis_index("c")
    s = lax.axis_index("s")
    wid = c * NS + s
    cbase = wid * NCHUNKS

    ebuf = [eb0, eb1, eb2, eb3]
    didx = [di0, di1, di2, di3]
    esg = [eg0, eg1, eg2, eg3]
    edg = [dg0, dg1, dg2, dg3]
    wbuf = [wb0, wb1, wb2, wb3]
    rows = [rw0, rw1, rw2, rw3]
    isem = [is0, is1, is2, is3]
    gsem = [gs0, gs1, gs2, gs3]
    esem = [es0, es1, es2, es3]
    ssem = [ss0, ss1, ss2, ss3]

    pltpu.sync_copy(mx_hbm, mx_v)

    def zfill(i, _):
        for h in range(8):
            rw0[i, pl.ds(h * 16, 16)] = jnp.zeros((16,), jnp.float32)
        return 0

    lax.fori_loop(0, CHUNK, zfill, 0)
    for g in range(GROUPS):
        wb0[pl.ds(g * 16, 16)] = jnp.zeros((16,), jnp.float32)

    for k in range(8):
        ci = s + NS * k

        @pl.when(ci < NZCH)
        def _():
            pltpu.sync_copy(rw0, u_acc.at[pl.ds(ci * CHUNK, CHUNK)])
            pltpu.sync_copy(wb0, den_acc.at[pl.ds(ci * CHUNK, CHUNK)])

    plsc.subcore_barrier()

    mx = mx_v[...]

    def idx_start(j, b):
        pltpu.async_copy(ep_hbm.at[pl.ds((cbase + j) * 2 * CHUNK, 2 * CHUNK)],
                         ebuf[b], isem[b])

    def idx_wait(j, b):
        pltpu.make_async_copy(
            ep_hbm.at[pl.ds((cbase + j) * 2 * CHUNK, 2 * CHUNK)],
            ebuf[b], isem[b]).wait()

    def gathers_start(b):
        src_ix = ebuf[b].at[pl.ds(0, CHUNK)]
        dst_ix = ebuf[b].at[pl.ds(CHUNK, CHUNK)]
        pltpu.async_copy(es_hbm.at[src_ix], esg[b], esem[b])
        pltpu.async_copy(ed_hbm.at[dst_ix], edg[b], esem[b])
        pltpu.async_copy(zs_hbm.at[src_ix], rows[b], gsem[b])

    def esed_wait(b):
        src_ix = ebuf[b].at[pl.ds(0, CHUNK)]
        dst_ix = ebuf[b].at[pl.ds(CHUNK, CHUNK)]
        pltpu.make_async_copy(es_hbm.at[src_ix], esg[b], esem[b]).wait()
        pltpu.make_async_copy(ed_hbm.at[dst_ix], edg[b], esem[b]).wait()

    def rows_wait(b):
        src_ix = ebuf[b].at[pl.ds(0, CHUNK)]
        pltpu.make_async_copy(zs_hbm.at[src_ix], rows[b], gsem[b]).wait()

    def scatter_start(b):
        pltpu.async_copy(rows[b], u_acc.at[didx[b]], ssem[b], add=True)
        pltpu.async_copy(wbuf[b], den_acc.at[didx[b]], ssem[b], add=True)

    def scatter_wait(b):
        pltpu.make_async_copy(rows[b], u_acc.at[didx[b]], ssem[b]).wait()
        pltpu.make_async_copy(wbuf[b], den_acc.at[didx[b]], ssem[b]).wait()

    def process(b):
        esed_wait(b)
        for g in range(GROUPS):
            didx[b][pl.ds(g * 16, 16)] = ebuf[b][pl.ds(CHUNK + g * 16, 16)]
            es_s = esg[b][pl.ds(g * 16, 16)]
            ed_d = edg[b][pl.ds(g * 16, 16)]
            w = jnp.exp(_lrelu(es_s - ed_d) - _lrelu(mx - ed_d))
            wbuf[b][pl.ds(g * 16, 16)] = w
        rows_wait(b)

        @functools.partial(plsc.parallel_loop, 0, CHUNK, unroll=2)
        def _(i):
            wv = plsc.load_gather(wbuf[b], [jnp.broadcast_to(i, (16,))])
            for h in range(8):
                rows[b][i, pl.ds(h * 16, 16)] = rows[b][i, pl.ds(h * 16, 16)] * wv

        scatter_start(b)

    for j in range(3):
        idx_start(j, j)
        idx_wait(j, j)
    gathers_start(0)
    gathers_start(1)

    def quad(t, _):
        for i in range(4):
            jc = 4 * t + i
            process(i)

            @pl.when(jc >= 1)
            def _():
                scatter_wait((i + 3) % 4)

            @pl.when(jc <= NCHUNKS - 4)
            def _():
                idx_start(jc + 3, (i + 3) % 4)

            @pl.when(jc <= NCHUNKS - 3)
            def _():
                @pl.when(jc >= 1)
                def _():
                    idx_wait(jc + 2, (i + 2) % 4)

                gathers_start((i + 2) % 4)
        return 0

    lax.fori_loop(0, (NCHUNKS - 1) // 4, quad, 0)

    process(0)
    scatter_wait(3)
    scatter_wait(0)

    plsc.subcore_barrier()

    for k in range(8):
        ci = s + NS * k

        @pl.when(ci < NZCH)
        def _():
            pltpu.sync_copy(u_acc.at[pl.ds(ci * CHUNK, CHUNK)], rw0)
            pltpu.sync_copy(rw0, u_out.at[pl.ds(c * N + ci * CHUNK, CHUNK)])
            pltpu.sync_copy(den_acc.at[pl.ds(ci * CHUNK, CHUNK)], wb0)
            pltpu.sync_copy(wb0, den_out.at[pl.ds(c * N + ci * CHUNK, CHUNK)])


_edge_phase = functools.partial(
    pl.kernel,
    out_type=[
        jax.ShapeDtypeStruct((NC * N, H), jnp.float32),
        jax.ShapeDtypeStruct((NC * N,), jnp.float32),
    ],
    mesh=plsc.VectorSubcoreMesh(core_axis_name="c", subcore_axis_name="s"),
    compiler_params=pltpu.CompilerParams(needs_layout_passes=False),
    scratch_types=(
        [pltpu.VMEM((2 * CHUNK,), jnp.int32) for _ in range(4)]
        + [pltpu.VMEM((CHUNK,), jnp.int32) for _ in range(4)]
        + [pltpu.VMEM((CHUNK,), jnp.float32) for _ in range(4)]
        + [pltpu.VMEM((CHUNK,), jnp.float32) for _ in range(4)]
        + [pltpu.VMEM((CHUNK,), jnp.float32) for _ in range(4)]
        + [pltpu.VMEM((CHUNK, H), jnp.float32) for _ in range(4)]
        + [
            pltpu.VMEM((16,), jnp.float32),
            pltpu.VMEM_SHARED((N, H), jnp.float32),
            pltpu.VMEM_SHARED((N,), jnp.float32),
        ]
        + [pltpu.SemaphoreType.DMA] * 16
    ),
)(_sc_body)


def _gru_body(u_ref, den_ref, wz_ref, wn_ref, wo_ref, out_ref):
    usum = u_ref[0] + u_ref[1]
    den = den_ref[0] + den_ref[1]
    h_k = usum * (1.0 / jnp.maximum(den, 1e-30))
    zg = jax.nn.sigmoid(jnp.dot(h_k, wz_ref[...], preferred_element_type=jnp.float32))
    ng = jnp.tanh(jnp.dot(h_k, wn_ref[...], preferred_element_type=jnp.float32))
    out_ref[...] = jnp.dot((1.0 - zg) * ng, wo_ref[...],
                           preferred_element_type=jnp.float32)


def _gru(u, den, WzT, WnT, WoT):
    return pl.pallas_call(
        _gru_body,
        grid=(N // B1,),
        in_specs=[
            pl.BlockSpec((NC, B1, H), lambda i: (0, i, 0)),
            pl.BlockSpec((NC, B1, 1), lambda i: (0, i, 0)),
            pl.BlockSpec((H, H), lambda i: (0, 0)),
            pl.BlockSpec((H, H), lambda i: (0, 0)),
            pl.BlockSpec((H, H), lambda i: (0, 0)),
        ],
        out_specs=pl.BlockSpec((B1, H), lambda i: (i, 0)),
        out_shape=jax.ShapeDtypeStruct((N, H), jnp.float32),
    )(u, den, WzT, WnT, WoT)


def kernel(h_src, h_dst, edge_index, W_src, W_dst, a, W_ih, W_hh, W_out):
    ei = edge_index.astype(jnp.int32)
    src3 = ei[0].reshape(NW, NCHUNKS, CHUNK)
    dst3 = ei[1].reshape(NW, NCHUNKS, CHUNK)
    epack = jnp.stack([src3, dst3], axis=2).reshape(E * 2)
    a2 = a.reshape(H, 1)
    zs, es2, ed2, mx = _prep(h_src, h_dst, W_src, W_dst, a2)
    es = es2.reshape(N)
    ed = ed2.reshape(N)
    mx16 = jnp.broadcast_to(mx.reshape(1), (16,))
    u, den = _edge_phase(zs, es, ed, mx16, epack)
    WzT = W_ih[H:2 * H].T
    WnT = W_ih[2 * H:3 * H].T
    WoT = W_out.T
    return _gru(u.reshape(NC, N, H), den.reshape(NC, N, 1), WzT, WnT, WoT)
